# baseline (speedup 1.0000x reference)
import functools

import jax
import jax.numpy as jnp
from jax import lax
from jax.experimental import pallas as pl
from jax.experimental.pallas import tpu as pltpu
from jax.experimental.pallas import tpu_sc as plsc

_B, _S, _D = 4, 4096, 768
_A, _C = 8, 2
_K = 64
_TS = 512
_ST = _S // _TS
_SA = _S * _A
_NVROW = _SA // 16
_LDUMP = 80
_LSZ = 96


def _proj_kernel(x_ref, lab_ref, wr_ref, br_ref,
                 predict_ref, scores_ref, logits_ref):
    x = x_ref[0]
    logits = jnp.dot(x, wr_ref[...], preferred_element_type=jnp.float32) + br_ref[...]
    l0 = logits[:, :_A]
    l1 = logits[:, _A:]
    predict_ref[0] = jnp.where(l1 > l0, jnp.int32(1), jnp.int32(0))
    scores_ref[0] = jnp.where(lab_ref[0] == 1, jnp.float32(2.0), jax.nn.sigmoid(l1))
    logits_ref[0] = jnp.concatenate(
        [logits, jnp.zeros((_TS, 128 - 2 * _A), jnp.float32)], axis=1)


def _run_stage1(batch_input, anchor_labels, wr, br):
    return pl.pallas_call(
        _proj_kernel,
        grid=(_B, _ST),
        in_specs=[
            pl.BlockSpec((1, _TS, _D), lambda b_, s_: (b_, s_, 0)),
            pl.BlockSpec((1, _TS, _A), lambda b_, s_: (b_, s_, 0)),
            pl.BlockSpec((_D, 2 * _A), lambda b_, s_: (0, 0)),
            pl.BlockSpec((1, 2 * _A), lambda b_, s_: (0, 0)),
        ],
        out_specs=[
            pl.BlockSpec((1, _TS, _A), lambda b_, s_: (b_, s_, 0)),
            pl.BlockSpec((1, _TS, _A), lambda b_, s_: (b_, s_, 0)),
            pl.BlockSpec((1, _TS, 128), lambda b_, s_: (b_, s_, 0)),
        ],
        out_shape=(
            jax.ShapeDtypeStruct((_B, _S, _A), jnp.int32),
            jax.ShapeDtypeStruct((_B, _S, _A), jnp.float32),
            jax.ShapeDtypeStruct((_B, _S, 128), jnp.float32),
        ),
        compiler_params=pltpu.CompilerParams(
            dimension_semantics=("parallel", "parallel")),
    )(batch_input, anchor_labels, wr, br)


def _sel_kernel(labels_hbm, scores_hbm, logits_hbm,
                idx_hbm, cand_hbm, lsum_hbm,
                lrow_v, list_v, tmp_v,
                selpos_v, sellab_v, gidx_v, rows_v,
                brow_v, srow_v, arow_v, candrow_v, lsumrow_v,
                slowbuf_v,
                dma_sem):
    cid = lax.axis_index("c")
    sid = lax.axis_index("s")
    batch = cid * 2 + sid
    ii = lax.broadcasted_iota(jnp.int32, (16,), 0)
    zi = jnp.zeros((16,), jnp.int32)
    fifteen = jnp.full((16,), 15, jnp.int32)
    kb = jnp.full((16,), _K, jnp.int32)
    dumpv = jnp.full((16,), _LDUMP, jnp.int32)

    @pl.when(sid < _B // 2)
    def _work():
        pltpu.sync_copy(labels_hbm.at[pl.ds(batch * _SA, _SA)], lrow_v)

        def _compact(g, cursorv):
            v = lrow_v[pl.ds(g * 16, 16)]
            s = v
            for d in (1, 2, 4, 8):
                tmp_v[...] = s
                gsh = plsc.load_gather(tmp_v, [jnp.maximum(ii - d, 0)])
                s = s + jnp.where(ii >= d, gsh, 0)
            tmp_v[...] = s
            totv = plsc.load_gather(tmp_v, [fifteen])
            mskb = v == 1
            rank = cursorv + s - 1
            idx = jnp.where(mskb & (rank < kb), rank, dumpv)
            plsc.store_scatter(list_v, [idx], g * 16 + ii)
            return cursorv + totv

        cntv = lax.fori_loop(0, _NVROW, _compact, zi)
        total_s = cntv[0]

        @pl.when(total_s >= _K)
        def _fast():
            one = jnp.full((16,), 1, jnp.int32)
            for g in range(_K // 16):
                selpos_v[pl.ds(g * 16, 16)] = list_v[pl.ds(g * 16, 16)]
                sellab_v[pl.ds(g * 16, 16)] = one

        @pl.when(total_s < _K)
        def _slow():
            pltpu.sync_copy(scores_hbm.at[pl.ds(batch * _SA, _SA)], slowbuf_v)

            def _ins4(p0, p1, p2, p3, j, val):
                hit = ii == jnp.bitwise_and(j, jnp.int32(15))
                g = j >> 4
                return (jnp.where(hit & (g == 0), val, p0),
                        jnp.where(hit & (g == 1), val, p1),
                        jnp.where(hit & (g == 2), val, p2),
                        jnp.where(hit & (g == 3), val, p3))

            def _one(j, carry):
                s0, s1, s2, s3, l0, l1, l2, l3 = carry

                def _scan(g, car):
                    av, ap = car
                    v = slowbuf_v[pl.ds(g * 16, 16)]
                    p = g * 16 + ii
                    take = v > av
                    return (jnp.where(take, v, av), jnp.where(take, p, ap))

                av, ap = lax.fori_loop(
                    0, _NVROW, _scan,
                    (jnp.full((16,), -2.0, jnp.float32),
                     jnp.zeros((16,), jnp.int32)))
                m = av[0]
                pos = ap[0]
                for lane in range(1, 16):
                    x = av[lane]
                    p = ap[lane]
                    better = (x > m) | ((x == m) & (p < pos))
                    m = jnp.where(better, x, m)
                    pos = jnp.where(better, p, pos)
                vg = pos >> 4
                lane = jnp.bitwise_and(pos, jnp.int32(15))
                vv = slowbuf_v[pl.ds(vg * 16, 16)]
                slowbuf_v[pl.ds(vg * 16, 16)] = jnp.where(
                    ii == lane, jnp.float32(-1.0), vv)
                lab = jnp.where(m == jnp.float32(2.0), jnp.int32(1), jnp.int32(0))
                s0, s1, s2, s3 = _ins4(s0, s1, s2, s3, j, pos)
                l0, l1, l2, l3 = _ins4(l0, l1, l2, l3, j, lab)
                return s0, s1, s2, s3, l0, l1, l2, l3

            s0, s1, s2, s3, l0, l1, l2, l3 = lax.fori_loop(
                0, _K, _one, (zi, zi, zi, zi, zi, zi, zi, zi))
            selpos_v[pl.ds(0, 16)] = s0
            selpos_v[pl.ds(16, 16)] = s1
            selpos_v[pl.ds(32, 16)] = s2
            selpos_v[pl.ds(48, 16)] = s3
            sellab_v[pl.ds(0, 16)] = l0
            sellab_v[pl.ds(16, 16)] = l1
            sellab_v[pl.ds(32, 16)] = l2
            sellab_v[pl.ds(48, 16)] = l3

        for g in range(_K // 16):
            pos = selpos_v[pl.ds(g * 16, 16)]
            s = pos >> 3
            a = jnp.bitwise_and(pos, jnp.int32(7))
            srow_v[pl.ds(g * 16, 16)] = s
            arow_v[pl.ds(g * 16, 16)] = a
            brow_v[pl.ds(g * 16, 16)] = jnp.full((16,), batch, jnp.int32)
            gidx_v[pl.ds(g * 16, 16)] = batch * _S + s

        pltpu.async_copy(logits_hbm.at[gidx_v], rows_v, dma_sem).wait()

        msum = jnp.zeros((16,), jnp.float32)
        for g in range(_K // 16):
            ridx = g * 16 + ii
            a = arow_v[pl.ds(g * 16, 16)]
            lab = sellab_v[pl.ds(g * 16, 16)]
            x0 = plsc.load_gather(rows_v, [ridx, a])
            x1 = plsc.load_gather(rows_v, [ridx, a + 8])
            xy = jnp.where(lab == 1, x1, x0)
            xo = jnp.where(lab == 1, x0, x1)
            msum = msum + jnp.maximum(jnp.float32(0.0),
                                      jnp.float32(5.0) - xy + xo) * jnp.float32(0.5)
            candrow_v[pl.ds(g * 16, 16)] = jnp.where(
                x1 > x0, jnp.int32(1), jnp.int32(0))

        lsumrow_v[...] = msum

        pltpu.sync_copy(brow_v, idx_hbm.at[pl.ds(batch * 3 * _K, _K)])
        pltpu.sync_copy(srow_v, idx_hbm.at[pl.ds(batch * 3 * _K + _K, _K)])
        pltpu.sync_copy(arow_v, idx_hbm.at[pl.ds(batch * 3 * _K + 2 * _K, _K)])
        pltpu.sync_copy(candrow_v, cand_hbm.at[pl.ds(batch * _K, _K)])
        pltpu.sync_copy(lsumrow_v, lsum_hbm.at[pl.ds(batch * 16, 16)])


def _run_stage2(labels_flat, scores_flat, logits2):
    mesh = plsc.VectorSubcoreMesh(core_axis_name="c", subcore_axis_name="s")
    kfn = functools.partial(
        pl.kernel,
        mesh=mesh,
        compiler_params=pltpu.CompilerParams(needs_layout_passes=False),
        out_type=[
            jax.ShapeDtypeStruct((_B * 3 * _K,), jnp.int32),
            jax.ShapeDtypeStruct((_B * _K,), jnp.int32),
            jax.ShapeDtypeStruct((_B * 16,), jnp.float32),
        ],
        scratch_types=[
            pltpu.VMEM((_SA,), jnp.int32),
            pltpu.VMEM((_LSZ,), jnp.int32),
            pltpu.VMEM((16,), jnp.int32),
            pltpu.VMEM((_K,), jnp.int32),
            pltpu.VMEM((_K,), jnp.int32),
            pltpu.VMEM((_K,), jnp.int32),
            pltpu.VMEM((_K, 128), jnp.float32),
            pltpu.VMEM((_K,), jnp.int32),
            pltpu.VMEM((_K,), jnp.int32),
            pltpu.VMEM((_K,), jnp.int32),
            pltpu.VMEM((_K,), jnp.int32),
            pltpu.VMEM((16,), jnp.float32),
            pltpu.VMEM((_SA,), jnp.float32),
            pltpu.SemaphoreType.DMA,
        ],
    )(_sel_kernel)
    return kfn(labels_flat, scores_flat, logits2)


def _loss_kernel(ls_ref, loss_ref):
    loss_ref[...] = jnp.full(
        (1, 1), jnp.sum(ls_ref[...]) * jnp.float32(1.0 / (_B * _K)))


def _run_stage3(lsum):
    return pl.pallas_call(
        _loss_kernel,
        out_shape=jax.ShapeDtypeStruct((1, 1), jnp.float32),
    )(lsum.reshape(_B, 16))


def kernel(batch_input, anchor_labels, W, b):
    w0 = W[0::2]
    w1 = W[1::2]
    wr = jnp.concatenate([w0, w1], axis=0).T
    br = jnp.concatenate([b[0::2], b[1::2]]).reshape(1, 2 * _A)

    predict, scores3, logits3 = _run_stage1(batch_input, anchor_labels, wr, br)

    labels_flat = anchor_labels.reshape(_B * _SA)
    scores_flat = scores3.reshape(_B * _SA)
    logits2 = logits3.reshape(_B * _S, 128)
    idx_flat, cand_flat, lsum = _run_stage2(labels_flat, scores_flat, logits2)

    loss = _run_stage3(lsum)[0, 0]
    total_idx = idx_flat.reshape(_B, 3, _K).transpose(0, 2, 1).reshape(_B * _K, 3)
    candidate_label = cand_flat
    return loss, predict, total_idx, candidate_label


# speedup vs baseline: 1.4606x; 1.4606x over previous
import functools

import jax
import jax.numpy as jnp
from jax import lax
from jax.experimental import pallas as pl
from jax.experimental.pallas import tpu as pltpu
from jax.experimental.pallas import tpu_sc as plsc

_B, _S, _D = 4, 4096, 768
_A, _C = 8, 2
_K = 64
_TS = 512
_ST = _S // _TS
_SA = _S * _A
_NVROW = _SA // 16
_LDUMP = 80
_LSZ = 96


def _proj_kernel(x_ref, lab_ref, wr_ref, br_ref,
                 predict_ref, scores_ref, logits_ref):
    x = x_ref[0]
    logits = jnp.dot(x, wr_ref[...], preferred_element_type=jnp.float32) + br_ref[...]
    l0 = logits[:, :_A]
    l1 = logits[:, _A:]
    predict_ref[0] = jnp.where(l1 > l0, jnp.int32(1), jnp.int32(0))
    scores_ref[0] = jnp.where(lab_ref[0] == 1, jnp.float32(2.0), jax.nn.sigmoid(l1))
    logits_ref[0] = jnp.concatenate(
        [logits, jnp.zeros((_TS, 128 - 2 * _A), jnp.float32)], axis=1)


def _run_stage1(batch_input, anchor_labels, wr, br):
    return pl.pallas_call(
        _proj_kernel,
        grid=(_B, _ST),
        in_specs=[
            pl.BlockSpec((1, _TS, _D), lambda b_, s_: (b_, s_, 0)),
            pl.BlockSpec((1, _TS, _A), lambda b_, s_: (b_, s_, 0)),
            pl.BlockSpec((_D, 2 * _A), lambda b_, s_: (0, 0)),
            pl.BlockSpec((1, 2 * _A), lambda b_, s_: (0, 0)),
        ],
        out_specs=[
            pl.BlockSpec((1, _TS, _A), lambda b_, s_: (b_, s_, 0)),
            pl.BlockSpec((1, _TS, _A), lambda b_, s_: (b_, s_, 0)),
            pl.BlockSpec((1, _TS, 128), lambda b_, s_: (b_, s_, 0)),
        ],
        out_shape=(
            jax.ShapeDtypeStruct((_B, _S, _A), jnp.int32),
            jax.ShapeDtypeStruct((_B, _S, _A), jnp.float32),
            jax.ShapeDtypeStruct((_B, _S, 128), jnp.float32),
        ),
        compiler_params=pltpu.CompilerParams(
            dimension_semantics=("parallel", "parallel")),
    )(batch_input, anchor_labels, wr, br)


def _pick_kernel(labels_hbm, selpos_hbm, seltot_hbm,
                 lrow_v, list_v, tmp_v, cnt_v):
    cid = lax.axis_index("c")
    sid = lax.axis_index("s")
    batch = cid * 2 + sid
    ii = lax.broadcasted_iota(jnp.int32, (16,), 0)
    zi = jnp.zeros((16,), jnp.int32)
    fifteen = jnp.full((16,), 15, jnp.int32)
    kb = jnp.full((16,), _K, jnp.int32)
    dumpv = jnp.full((16,), _LDUMP, jnp.int32)

    @pl.when(sid < _B // 2)
    def _work():
        pltpu.sync_copy(labels_hbm.at[pl.ds(batch * _SA, _SA)], lrow_v)

        def _compact(g, cursorv):
            v = lrow_v[pl.ds(g * 16, 16)]
            s = v
            for d in (1, 2, 4, 8):
                tmp_v[...] = s
                gsh = plsc.load_gather(tmp_v, [jnp.maximum(ii - d, 0)])
                s = s + jnp.where(ii >= d, gsh, 0)
            tmp_v[...] = s
            totv = plsc.load_gather(tmp_v, [fifteen])
            mskb = v == 1
            rank = cursorv + s - 1
            idx = jnp.where(mskb & (rank < kb), rank, dumpv)
            plsc.store_scatter(list_v, [idx], g * 16 + ii)
            return cursorv + totv

        cntv = lax.fori_loop(0, _NVROW, _compact, zi)
        cnt_v[...] = cntv
        pltpu.sync_copy(list_v.at[pl.ds(0, _K)],
                        selpos_hbm.at[pl.ds(batch * _K, _K)])
        pltpu.sync_copy(cnt_v, seltot_hbm.at[pl.ds(batch * 16, 16)])


def _run_pick(labels_flat):
    mesh = plsc.VectorSubcoreMesh(core_axis_name="c", subcore_axis_name="s")
    kfn = functools.partial(
        pl.kernel,
        mesh=mesh,
        compiler_params=pltpu.CompilerParams(needs_layout_passes=False),
        out_type=[
            jax.ShapeDtypeStruct((_B * _K,), jnp.int32),
            jax.ShapeDtypeStruct((_B * 16,), jnp.int32),
        ],
        scratch_types=[
            pltpu.VMEM((_SA,), jnp.int32),
            pltpu.VMEM((_LSZ,), jnp.int32),
            pltpu.VMEM((16,), jnp.int32),
            pltpu.VMEM((16,), jnp.int32),
        ],
    )(_pick_kernel)
    return kfn(labels_flat)


def _sel_kernel(selpos_hbm, seltot_hbm, scores_hbm, logits_hbm,
                idx_hbm, cand_hbm, lsum_hbm,
                cnt_v,
                selpos_v, sellab_v, gidx_v, rows_v,
                brow_v, srow_v, arow_v, candrow_v, lsumrow_v,
                slowbuf_v,
                dma_sem):
    cid = lax.axis_index("c")
    sid = lax.axis_index("s")
    batch = cid * 2 + sid
    ii = lax.broadcasted_iota(jnp.int32, (16,), 0)
    zi = jnp.zeros((16,), jnp.int32)

    @pl.when(sid < _B // 2)
    def _work():
        pltpu.sync_copy(seltot_hbm.at[pl.ds(batch * 16, 16)], cnt_v)
        total_s = cnt_v[...][0]

        @pl.when(total_s >= _K)
        def _fast():
            pltpu.sync_copy(selpos_hbm.at[pl.ds(batch * _K, _K)], selpos_v)
            one = jnp.full((16,), 1, jnp.int32)
            for g in range(_K // 16):
                sellab_v[pl.ds(g * 16, 16)] = one

        @pl.when(total_s < _K)
        def _slow():
            pltpu.sync_copy(scores_hbm.at[pl.ds(batch * _SA, _SA)], slowbuf_v)

            def _ins4(p0, p1, p2, p3, j, val):
                hit = ii == jnp.bitwise_and(j, jnp.int32(15))
                g = j >> 4
                return (jnp.where(hit & (g == 0), val, p0),
                        jnp.where(hit & (g == 1), val, p1),
                        jnp.where(hit & (g == 2), val, p2),
                        jnp.where(hit & (g == 3), val, p3))

            def _one(j, carry):
                s0, s1, s2, s3, l0, l1, l2, l3 = carry

                def _scan(g, car):
                    av, ap = car
                    v = slowbuf_v[pl.ds(g * 16, 16)]
                    p = g * 16 + ii
                    take = v > av
                    return (jnp.where(take, v, av), jnp.where(take, p, ap))

                av, ap = lax.fori_loop(
                    0, _NVROW, _scan,
                    (jnp.full((16,), -2.0, jnp.float32),
                     jnp.zeros((16,), jnp.int32)))
                m = av[0]
                pos = ap[0]
                for lane in range(1, 16):
                    x = av[lane]
                    p = ap[lane]
                    better = (x > m) | ((x == m) & (p < pos))
                    m = jnp.where(better, x, m)
                    pos = jnp.where(better, p, pos)
                vg = pos >> 4
                lane = jnp.bitwise_and(pos, jnp.int32(15))
                vv = slowbuf_v[pl.ds(vg * 16, 16)]
                slowbuf_v[pl.ds(vg * 16, 16)] = jnp.where(
                    ii == lane, jnp.float32(-1.0), vv)
                lab = jnp.where(m == jnp.float32(2.0), jnp.int32(1), jnp.int32(0))
                s0, s1, s2, s3 = _ins4(s0, s1, s2, s3, j, pos)
                l0, l1, l2, l3 = _ins4(l0, l1, l2, l3, j, lab)
                return s0, s1, s2, s3, l0, l1, l2, l3

            s0, s1, s2, s3, l0, l1, l2, l3 = lax.fori_loop(
                0, _K, _one, (zi, zi, zi, zi, zi, zi, zi, zi))
            selpos_v[pl.ds(0, 16)] = s0
            selpos_v[pl.ds(16, 16)] = s1
            selpos_v[pl.ds(32, 16)] = s2
            selpos_v[pl.ds(48, 16)] = s3
            sellab_v[pl.ds(0, 16)] = l0
            sellab_v[pl.ds(16, 16)] = l1
            sellab_v[pl.ds(32, 16)] = l2
            sellab_v[pl.ds(48, 16)] = l3

        for g in range(_K // 16):
            pos = selpos_v[pl.ds(g * 16, 16)]
            s = pos >> 3
            a = jnp.bitwise_and(pos, jnp.int32(7))
            srow_v[pl.ds(g * 16, 16)] = s
            arow_v[pl.ds(g * 16, 16)] = a
            brow_v[pl.ds(g * 16, 16)] = jnp.full((16,), batch, jnp.int32)
            gidx_v[pl.ds(g * 16, 16)] = batch * _S + s

        pltpu.async_copy(logits_hbm.at[gidx_v], rows_v, dma_sem).wait()

        msum = jnp.zeros((16,), jnp.float32)
        for g in range(_K // 16):
            ridx = g * 16 + ii
            a = arow_v[pl.ds(g * 16, 16)]
            lab = sellab_v[pl.ds(g * 16, 16)]
            x0 = plsc.load_gather(rows_v, [ridx, a])
            x1 = plsc.load_gather(rows_v, [ridx, a + 8])
            xy = jnp.where(lab == 1, x1, x0)
            xo = jnp.where(lab == 1, x0, x1)
            msum = msum + jnp.maximum(jnp.float32(0.0),
                                      jnp.float32(5.0) - xy + xo) * jnp.float32(0.5)
            candrow_v[pl.ds(g * 16, 16)] = jnp.where(
                x1 > x0, jnp.int32(1), jnp.int32(0))

        lsumrow_v[...] = msum

        pltpu.sync_copy(brow_v, idx_hbm.at[pl.ds(batch * 3 * _K, _K)])
        pltpu.sync_copy(srow_v, idx_hbm.at[pl.ds(batch * 3 * _K + _K, _K)])
        pltpu.sync_copy(arow_v, idx_hbm.at[pl.ds(batch * 3 * _K + 2 * _K, _K)])
        pltpu.sync_copy(candrow_v, cand_hbm.at[pl.ds(batch * _K, _K)])
        pltpu.sync_copy(lsumrow_v, lsum_hbm.at[pl.ds(batch * 16, 16)])


def _run_stage2(selpos, seltot, scores_flat, logits2):
    mesh = plsc.VectorSubcoreMesh(core_axis_name="c", subcore_axis_name="s")
    kfn = functools.partial(
        pl.kernel,
        mesh=mesh,
        compiler_params=pltpu.CompilerParams(needs_layout_passes=False),
        out_type=[
            jax.ShapeDtypeStruct((_B * 3 * _K,), jnp.int32),
            jax.ShapeDtypeStruct((_B * _K,), jnp.int32),
            jax.ShapeDtypeStruct((_B * 16,), jnp.float32),
        ],
        scratch_types=[
            pltpu.VMEM((16,), jnp.int32),
            pltpu.VMEM((_K,), jnp.int32),
            pltpu.VMEM((_K,), jnp.int32),
            pltpu.VMEM((_K,), jnp.int32),
            pltpu.VMEM((_K, 128), jnp.float32),
            pltpu.VMEM((_K,), jnp.int32),
            pltpu.VMEM((_K,), jnp.int32),
            pltpu.VMEM((_K,), jnp.int32),
            pltpu.VMEM((_K,), jnp.int32),
            pltpu.VMEM((16,), jnp.float32),
            pltpu.VMEM((_SA,), jnp.float32),
            pltpu.SemaphoreType.DMA,
        ],
    )(_sel_kernel)
    return kfn(selpos, seltot, scores_flat, logits2)


def _loss_kernel(ls_ref, loss_ref):
    loss_ref[...] = jnp.full(
        (1, 1), jnp.sum(ls_ref[...]) * jnp.float32(1.0 / (_B * _K)))


def _run_stage3(lsum):
    return pl.pallas_call(
        _loss_kernel,
        out_shape=jax.ShapeDtypeStruct((1, 1), jnp.float32),
    )(lsum.reshape(_B, 16))


def kernel(batch_input, anchor_labels, W, b):
    w0 = W[0::2]
    w1 = W[1::2]
    wr = jnp.concatenate([w0, w1], axis=0).T
    br = jnp.concatenate([b[0::2], b[1::2]]).reshape(1, 2 * _A)

    labels_flat = anchor_labels.reshape(_B * _SA)
    selpos, seltot = _run_pick(labels_flat)

    predict, scores3, logits3 = _run_stage1(batch_input, anchor_labels, wr, br)

    scores_flat = scores3.reshape(_B * _SA)
    logits2 = logits3.reshape(_B * _S, 128)
    idx_flat, cand_flat, lsum = _run_stage2(selpos, seltot, scores_flat, logits2)

    loss = _run_stage3(lsum)[0, 0]
    total_idx = idx_flat.reshape(_B, 3, _K).transpose(0, 2, 1).reshape(_B * _K, 3)
    candidate_label = cand_flat
    return loss, predict, total_idx, candidate_label
